# baseline (speedup 1.0000x reference)
import functools

import jax
import jax.numpy as jnp
from jax.experimental import pallas as pl

_BM = 512


def _router_body(x_ref, wt_ref, logits_ref, w_ref, idx_ref, *, n_experts):
    l = jnp.dot(x_ref[:, :], wt_ref[:, :],
                preferred_element_type=jnp.float32)
    m = jnp.max(l, axis=1, keepdims=True)
    s = jnp.sum(jnp.exp(l - m), axis=1, keepdims=True)
    iota = jax.lax.broadcasted_iota(jnp.int32, l.shape, 1)
    idx = jnp.min(jnp.where(l == m, iota, n_experts), axis=1, keepdims=True)
    logits_ref[:, :] = l
    w_ref[:, :] = 1.0 / s
    idx_ref[:, :] = idx


def kernel(hidden_states, W):
    b, s, h = hidden_states.shape
    e = W.shape[0]
    n = b * s
    x = hidden_states.reshape(n, h)
    wt = W.T

    grid = (n // _BM,)
    logits, weights, indices = pl.pallas_call(
        functools.partial(_router_body, n_experts=e),
        grid=grid,
        in_specs=[
            pl.BlockSpec((_BM, h), lambda i: (i, 0)),
            pl.BlockSpec((h, e), lambda i: (0, 0)),
        ],
        out_specs=[
            pl.BlockSpec((_BM, e), lambda i: (i, 0)),
            pl.BlockSpec((_BM, 1), lambda i: (i, 0)),
            pl.BlockSpec((_BM, 1), lambda i: (i, 0)),
        ],
        out_shape=[
            jax.ShapeDtypeStruct((n, e), jnp.float32),
            jax.ShapeDtypeStruct((n, 1), jnp.float32),
            jax.ShapeDtypeStruct((n, 1), jnp.int32),
        ],
    )(x, wt)

    return (weights.reshape(b, s, 1),
            indices.reshape(b, s, 1),
            logits.reshape(b, s, e))


# speedup vs baseline: 1.0151x; 1.0151x over previous
import functools

import jax
import jax.numpy as jnp
from jax.experimental import pallas as pl

_BM = 1024


def _router_body(x_ref, wt_ref, logits_ref, w_ref, idx_ref, *, n_experts):
    l = jnp.dot(x_ref[:, :], wt_ref[:, :],
                preferred_element_type=jnp.float32)
    m = jnp.max(l, axis=1, keepdims=True)
    s = jnp.sum(jnp.exp(l - m), axis=1, keepdims=True)
    iota = jax.lax.broadcasted_iota(jnp.int32, l.shape, 1)
    idx = jnp.min(jnp.where(l == m, iota, n_experts), axis=1, keepdims=True)
    logits_ref[:, :] = l
    w_ref[:, :] = 1.0 / s
    idx_ref[:, :] = idx


def kernel(hidden_states, W):
    b, s, h = hidden_states.shape
    e = W.shape[0]
    n = b * s
    x = hidden_states.reshape(n, h)
    wt = W.T

    grid = (n // _BM,)
    logits, weights, indices = pl.pallas_call(
        functools.partial(_router_body, n_experts=e),
        grid=grid,
        in_specs=[
            pl.BlockSpec((_BM, h), lambda i: (i, 0)),
            pl.BlockSpec((h, e), lambda i: (0, 0)),
        ],
        out_specs=[
            pl.BlockSpec((_BM, e), lambda i: (i, 0)),
            pl.BlockSpec((_BM, 1), lambda i: (i, 0)),
            pl.BlockSpec((_BM, 1), lambda i: (i, 0)),
        ],
        out_shape=[
            jax.ShapeDtypeStruct((n, e), jnp.float32),
            jax.ShapeDtypeStruct((n, 1), jnp.float32),
            jax.ShapeDtypeStruct((n, 1), jnp.int32),
        ],
    )(x, wt)

    return (weights.reshape(b, s, 1),
            indices.reshape(b, s, 1),
            logits.reshape(b, s, e))


# speedup vs baseline: 1.0158x; 1.0007x over previous
import functools

import jax
import jax.numpy as jnp
from jax.experimental import pallas as pl
from jax.experimental.pallas import tpu as pltpu

_BM = 1024


def _router_body(x_ref, wt_ref, logits_ref, w_ref, idx_ref, *, n_experts):
    l = jnp.dot(x_ref[:, :], wt_ref[:, :],
                preferred_element_type=jnp.float32)
    m = jnp.max(l, axis=1, keepdims=True)
    s = jnp.sum(jnp.exp(l - m), axis=1, keepdims=True)
    iota = jax.lax.broadcasted_iota(jnp.int32, l.shape, 1)
    idx = jnp.min(jnp.where(l == m, iota, n_experts), axis=1, keepdims=True)
    logits_ref[:, :] = l
    w_ref[:, :] = 1.0 / s
    idx_ref[:, :] = idx


def kernel(hidden_states, W):
    b, s, h = hidden_states.shape
    e = W.shape[0]
    n = b * s
    x = hidden_states.reshape(n, h)
    wt = W.T

    grid = (n // _BM,)
    logits, weights, indices = pl.pallas_call(
        functools.partial(_router_body, n_experts=e),
        grid=grid,
        in_specs=[
            pl.BlockSpec((_BM, h), lambda i: (i, 0)),
            pl.BlockSpec((h, e), lambda i: (0, 0)),
        ],
        out_specs=[
            pl.BlockSpec((_BM, e), lambda i: (i, 0)),
            pl.BlockSpec((_BM, 1), lambda i: (i, 0)),
            pl.BlockSpec((_BM, 1), lambda i: (i, 0)),
        ],
        out_shape=[
            jax.ShapeDtypeStruct((n, e), jnp.float32),
            jax.ShapeDtypeStruct((n, 1), jnp.float32),
            jax.ShapeDtypeStruct((n, 1), jnp.int32),
        ],
        compiler_params=pltpu.CompilerParams(
            dimension_semantics=("parallel",),
        ),
    )(x, wt)

    return (weights.reshape(b, s, 1),
            indices.reshape(b, s, 1),
            logits.reshape(b, s, e))


# speedup vs baseline: 1.0159x; 1.0001x over previous
import functools

import jax
import jax.numpy as jnp
from jax.experimental import pallas as pl
from jax.experimental.pallas import tpu as pltpu

_BM = 1024
_NK = 4


def _router_body(*refs, n_experts, nk):
    x_refs = refs[:nk]
    wt_refs = refs[nk:2 * nk]
    logits_ref, w_ref, idx_ref = refs[2 * nk:]
    l = jnp.dot(x_refs[0][:, :], wt_refs[0][:, :],
                preferred_element_type=jnp.float32)
    for k in range(1, nk):
        l = l + jnp.dot(x_refs[k][:, :], wt_refs[k][:, :],
                        preferred_element_type=jnp.float32)
    m = jnp.max(l, axis=1, keepdims=True)
    s = jnp.sum(jnp.exp(l - m), axis=1, keepdims=True)
    iota = jax.lax.broadcasted_iota(jnp.int32, l.shape, 1)
    idx = jnp.min(jnp.where(l == m, iota, n_experts), axis=1, keepdims=True)
    logits_ref[:, :] = l
    w_ref[:, :] = 1.0 / s
    idx_ref[:, :] = idx


def kernel(hidden_states, W):
    b, s, h = hidden_states.shape
    e = W.shape[0]
    n = b * s
    x = hidden_states.reshape(n, h)
    wt = W.T
    hk = h // _NK

    grid = (n // _BM,)
    x_specs = [
        pl.BlockSpec((_BM, hk), functools.partial(lambda i, k: (i, k), k=k))
        for k in range(_NK)
    ]
    wt_specs = [
        pl.BlockSpec((hk, e), functools.partial(lambda i, k: (k, 0), k=k))
        for k in range(_NK)
    ]
    logits, weights, indices = pl.pallas_call(
        functools.partial(_router_body, n_experts=e, nk=_NK),
        grid=grid,
        in_specs=x_specs + wt_specs,
        out_specs=[
            pl.BlockSpec((_BM, e), lambda i: (i, 0)),
            pl.BlockSpec((_BM, 1), lambda i: (i, 0)),
            pl.BlockSpec((_BM, 1), lambda i: (i, 0)),
        ],
        out_shape=[
            jax.ShapeDtypeStruct((n, e), jnp.float32),
            jax.ShapeDtypeStruct((n, 1), jnp.float32),
            jax.ShapeDtypeStruct((n, 1), jnp.int32),
        ],
        compiler_params=pltpu.CompilerParams(
            dimension_semantics=("parallel",),
        ),
    )(*([x] * _NK + [wt] * _NK))

    return (weights.reshape(b, s, 1),
            indices.reshape(b, s, 1),
            logits.reshape(b, s, e))
